# baseline (speedup 1.0000x reference)
import functools

import jax
import jax.numpy as jnp
from jax import lax
from jax.experimental import pallas as pl
from jax.experimental.pallas import tpu as pltpu
from jax.experimental.pallas import tpu_sc as plsc

N = 10000
E = 320000
D = 128

NC = 2
NS = 16
NW = NC * NS
EPW = E // NW

RPW = 160
EP = NS * RPW * 128
NH = N // 2
ACCR = 5120
CR = 32
ZR = 160

_mesh = plsc.VectorSubcoreMesh(core_axis_name="c", subcore_axis_name="s")

_sc_params = pltpu.CompilerParams(needs_layout_passes=False)


def _worker_id():
    return lax.axis_index("s") * NC + lax.axis_index("c")


def _splat(v16, lane):
    idx = jnp.full((16, 1), lane, jnp.int32)
    return lax.gather(
        v16, idx,
        lax.GatherDimensionNumbers(
            offset_dims=(), collapsed_slice_dims=(0,), start_index_map=(0,)),
        (1,), mode=lax.GatherScatterMode.PROMISE_IN_BOUNDS)


@functools.partial(
    pl.kernel,
    out_type=jax.ShapeDtypeStruct((NW, N), jnp.float32),
    mesh=_mesh,
    compiler_params=_sc_params,
    scratch_types=[
        pltpu.VMEM((EPW,), jnp.int32),
        pltpu.VMEM((EPW,), jnp.float32),
        pltpu.VMEM((N,), jnp.float32),
    ],
)
def _deg_kernel(dst_hbm, ew_hbm, out_hbm, dst_v, ew_v, deg_v):
    w = _worker_id()
    base = w * EPW
    pltpu.sync_copy(dst_hbm.at[pl.ds(base, EPW)], dst_v)
    pltpu.sync_copy(ew_hbm.at[pl.ds(base, EPW)], ew_v)

    zeros = jnp.zeros((16,), jnp.float32)

    def zbody(i, carry):
        deg_v[pl.ds(i * 16, 16)] = zeros
        return carry

    lax.fori_loop(0, N // 16, zbody, 0)

    def ebody(g, carry):
        o = g * 16
        d = dst_v[pl.ds(o, 16)]
        ew = ew_v[pl.ds(o, 16)]
        plsc.addupdate_scatter(deg_v, [d], ew)
        return carry

    lax.fori_loop(0, EPW // 16, ebody, 0)
    pltpu.sync_copy(deg_v, out_hbm.at[w])


@functools.partial(
    pl.kernel,
    out_type=jax.ShapeDtypeStruct((E,), jnp.float32),
    mesh=_mesh,
    compiler_params=_sc_params,
    scratch_types=[
        pltpu.VMEM((EPW,), jnp.int32),
        pltpu.VMEM((EPW,), jnp.int32),
        pltpu.VMEM((EPW,), jnp.float32),
        pltpu.VMEM((EPW,), jnp.float32),
        pltpu.VMEM((N,), jnp.float32),
    ],
)
def _norm_kernel(src_hbm, dst_hbm, ew_hbm, dinv_hbm, out_hbm,
                 src_v, dst_v, ew_v, nrm_v, dinv_v):
    w = _worker_id()
    base = w * EPW
    pltpu.sync_copy(src_hbm.at[pl.ds(base, EPW)], src_v)
    pltpu.sync_copy(dst_hbm.at[pl.ds(base, EPW)], dst_v)
    pltpu.sync_copy(ew_hbm.at[pl.ds(base, EPW)], ew_v)
    pltpu.sync_copy(dinv_hbm, dinv_v)

    def body(g, carry):
        o = g * 16
        s = src_v[pl.ds(o, 16)]
        d = dst_v[pl.ds(o, 16)]
        ew = ew_v[pl.ds(o, 16)]
        a = plsc.load_gather(dinv_v, [s])
        b = plsc.load_gather(dinv_v, [d])
        nrm_v[pl.ds(o, 16)] = a * ew * b
        return carry

    lax.fori_loop(0, EPW // 16, body, 0)
    pltpu.sync_copy(nrm_v, out_hbm.at[pl.ds(base, EPW)])


@functools.partial(
    pl.kernel,
    out_type=jax.ShapeDtypeStruct((N, D), jnp.float32),
    mesh=_mesh,
    compiler_params=_sc_params,
    scratch_types=[
        pltpu.VMEM((CR * 128,), jnp.int32),
        pltpu.VMEM((CR * 128,), jnp.int32),
        pltpu.VMEM((CR * 128,), jnp.float32),
        pltpu.VMEM((128, D), jnp.float32),
        pltpu.VMEM((128, D), jnp.float32),
        pltpu.VMEM((128,), jnp.int32),
        pltpu.VMEM((128,), jnp.int32),
        pltpu.VMEM((128,), jnp.int32),
        pltpu.VMEM((128,), jnp.int32),
        pltpu.VMEM_SHARED((ACCR, D), jnp.float32),
        pltpu.SemaphoreType.DMA,
        pltpu.SemaphoreType.DMA,
        pltpu.SemaphoreType.DMA,
        pltpu.SemaphoreType.DMA,
    ],
)
def _msgpass_kernel(hw_hbm, src_hbm, dst_hbm, nrm_hbm, out_hbm,
                    sbuf, dbuf, nbuf, rb0, rb1, se0, se1, de0, de1,
                    acc, g0, g1, s0, s1):
    c = lax.axis_index("c")
    s = lax.axis_index("s")

    rbufs = (rb0, rb1)
    gidx = (se0, se1)
    didx = (de0, de1)
    gsems = (g0, g1)
    ssems = (s0, s1)

    zeros = jnp.zeros((16,), jnp.float32)

    def zbody(i, carry):
        for k8 in range(8):
            rb0[i, pl.ds(k8 * 16, 16)] = zeros
        return carry

    lax.fori_loop(0, 128, zbody, 0)
    for j in range(3):
        pltpu.sync_copy(rb0, acc.at[pl.ds(s * 312 + j * 128, 128)])
    plsc.subcore_barrier()

    def gstart(r, k):
        for j8 in range(8):
            gidx[k][pl.ds(j8 * 16, 16)] = (
                sbuf[pl.ds(r * 128 + j8 * 16, 16)])
        pltpu.async_copy(hw_hbm.at[gidx[k]], rbufs[k], gsems[k])

    def gwait(k):
        pltpu.make_async_copy(hw_hbm.at[gidx[k]], rbufs[k], gsems[k]).wait()

    def sstart(r, k):
        for j8 in range(8):
            d = dbuf[pl.ds(r * 128 + j8 * 16, 16)]
            dloc = d - c * NH
            ok = (dloc >= 0) & (dloc < NH)
            didx[k][pl.ds(j8 * 16, 16)] = jnp.where(ok, dloc, NH + (d & 63))
        pltpu.async_copy(rbufs[k], acc.at[didx[k]], ssems[k], add=True)

    def swait(k):
        pltpu.make_async_copy(rbufs[k], acc.at[didx[k]], ssems[k]).wait()

    def scale(k, r):
        rb = rbufs[k]

        def bbody(b, carry):
            nv = nbuf[pl.ds(r * 128 + b * 16, 16)]
            e0 = b * 16
            for lane in range(16):
                sp = _splat(nv, lane)
                e = e0 + lane
                for k8 in range(8):
                    rb[e, pl.ds(k8 * 16, 16)] = rb[e, pl.ds(k8 * 16, 16)] * sp
            return carry

        lax.fori_loop(0, 8, bbody, 0)

    for cc in range(RPW // CR):
        cbase = (s * RPW + cc * CR) * 128
        pltpu.sync_copy(src_hbm.at[pl.ds(cbase, CR * 128)], sbuf)
        pltpu.sync_copy(dst_hbm.at[pl.ds(cbase, CR * 128)], dbuf)
        pltpu.sync_copy(nrm_hbm.at[pl.ds(cbase, CR * 128)], nbuf)

        for k in range(2):
            gstart(k, k)
        for k in range(2):
            gwait(k)
            scale(k, k)
            sstart(k, k)

        def mbody(r2, carry):
            b2 = r2 * 2
            for k in range(2):
                swait(k)
                gstart(b2 + k, k)
            for k in range(2):
                gwait(k)
                scale(k, b2 + k)
                sstart(b2 + k, k)
            return carry

        lax.fori_loop(1, CR // 2, mbody, 0)
        for k in range(2):
            swait(k)

    plsc.subcore_barrier()

    out_base = c * NH + s * 312
    pltpu.sync_copy(
        acc.at[pl.ds(s * 312, ZR)],
        out_hbm.at[pl.ds(out_base, ZR)])

    @pl.when(s < NS - 1)
    def _wb_small():
        pltpu.sync_copy(
            acc.at[pl.ds(s * 312 + ZR, 152)],
            out_hbm.at[pl.ds(out_base + ZR, 152)])

    @pl.when(s == NS - 1)
    def _wb_last():
        pltpu.sync_copy(
            acc.at[pl.ds(s * 312 + ZR, ZR)],
            out_hbm.at[pl.ds(out_base + ZR, ZR)])


def _prep_body(p_ref, dinv_ref, d2_ref):
    deg = jnp.sum(p_ref[...], axis=0, keepdims=True) + 1.0
    dinv = lax.rsqrt(deg)
    dinv_ref[...] = dinv
    d2_ref[...] = dinv * dinv


_prep = pl.pallas_call(
    _prep_body,
    out_shape=[
        jax.ShapeDtypeStruct((1, N), jnp.float32),
        jax.ShapeDtypeStruct((1, N), jnp.float32),
    ],
)


def _mm0_body(x_ref, w_ref, hw_ref):
    hw_ref[...] = jnp.dot(
        x_ref[...], w_ref[...], preferred_element_type=jnp.float32)


_mm0 = pl.pallas_call(
    _mm0_body,
    out_shape=jax.ShapeDtypeStruct((N, D), jnp.float32),
)


def _mmn_body(p_ref, hwp_ref, d2_ref, b_ref, w_ref, hw_ref):
    h = jnp.maximum(
        p_ref[...] + d2_ref[...] * hwp_ref[...] + b_ref[...], 0.0)
    hw_ref[...] = jnp.dot(h, w_ref[...], preferred_element_type=jnp.float32)


_mmn = pl.pallas_call(
    _mmn_body,
    out_shape=jax.ShapeDtypeStruct((N, D), jnp.float32),
)


def _final_body(p_ref, hwp_ref, d2_ref, b_ref, out_ref):
    out_ref[...] = p_ref[...] + d2_ref[...] * hwp_ref[...] + b_ref[...]


_final = pl.pallas_call(
    _final_body,
    out_shape=jax.ShapeDtypeStruct((N, D), jnp.float32),
)


def kernel(x, edge_index, edge_attr, W0, b0, W1, b1, W2, b2):
    src = edge_index[0]
    dst = edge_index[1]

    partials = _deg_kernel(dst, edge_attr)
    dinv2d, d2 = _prep(partials)
    nrm = _norm_kernel(src, dst, edge_attr, dinv2d.reshape(N))

    pad = EP - E
    src2 = jnp.concatenate([src, jnp.zeros((pad,), src.dtype)])
    dst2 = jnp.concatenate([dst, jnp.zeros((pad,), dst.dtype)])
    nrm2 = jnp.concatenate([nrm, jnp.zeros((pad,), nrm.dtype)])
    d2c = d2.reshape(N, 1)
    b0r = b0.reshape(1, D)
    b1r = b1.reshape(1, D)
    b2r = b2.reshape(1, D)

    hw0 = _mm0(x, W0)
    p = _msgpass_kernel(hw0, src2, dst2, nrm2)
    hw1 = _mmn(p, hw0, d2c, b0r, W1)
    p = _msgpass_kernel(hw1, src2, dst2, nrm2)
    hw2 = _mmn(p, hw1, d2c, b1r, W2)
    p = _msgpass_kernel(hw2, src2, dst2, nrm2)
    out = _final(p, hw2, d2c, b2r)
    return out.reshape(1, N, D)


# speedup vs baseline: 1.0102x; 1.0102x over previous
import functools

import jax
import jax.numpy as jnp
from jax import lax
from jax.experimental import pallas as pl
from jax.experimental.pallas import tpu as pltpu
from jax.experimental.pallas import tpu_sc as plsc

N = 10000
E = 320000
D = 128

NC = 2
NS = 16
NW = NC * NS
EPW = E // NW

RPW = 160
EP = NS * RPW * 128
NH = N // 2
ACCR = 5120
CR = 32
ZR = 160

_mesh = plsc.VectorSubcoreMesh(core_axis_name="c", subcore_axis_name="s")

_sc_params = pltpu.CompilerParams(needs_layout_passes=False)


def _worker_id():
    return lax.axis_index("s") * NC + lax.axis_index("c")


def _splat(v16, lane):
    idx = jnp.full((16, 1), lane, jnp.int32)
    return lax.gather(
        v16, idx,
        lax.GatherDimensionNumbers(
            offset_dims=(), collapsed_slice_dims=(0,), start_index_map=(0,)),
        (1,), mode=lax.GatherScatterMode.PROMISE_IN_BOUNDS)


@functools.partial(
    pl.kernel,
    out_type=jax.ShapeDtypeStruct((NW, N), jnp.float32),
    mesh=_mesh,
    compiler_params=_sc_params,
    scratch_types=[
        pltpu.VMEM((EPW,), jnp.int32),
        pltpu.VMEM((EPW,), jnp.float32),
        pltpu.VMEM((N,), jnp.float32),
    ],
)
def _deg_kernel(dst_hbm, ew_hbm, out_hbm, dst_v, ew_v, deg_v):
    w = _worker_id()
    base = w * EPW
    pltpu.sync_copy(dst_hbm.at[pl.ds(base, EPW)], dst_v)
    pltpu.sync_copy(ew_hbm.at[pl.ds(base, EPW)], ew_v)

    zeros = jnp.zeros((16,), jnp.float32)

    def zbody(i, carry):
        deg_v[pl.ds(i * 16, 16)] = zeros
        return carry

    lax.fori_loop(0, N // 16, zbody, 0)

    def ebody(g, carry):
        o = g * 16
        d = dst_v[pl.ds(o, 16)]
        ew = ew_v[pl.ds(o, 16)]
        plsc.addupdate_scatter(deg_v, [d], ew)
        return carry

    lax.fori_loop(0, EPW // 16, ebody, 0)
    pltpu.sync_copy(deg_v, out_hbm.at[w])


@functools.partial(
    pl.kernel,
    out_type=jax.ShapeDtypeStruct((E,), jnp.float32),
    mesh=_mesh,
    compiler_params=_sc_params,
    scratch_types=[
        pltpu.VMEM((EPW,), jnp.int32),
        pltpu.VMEM((EPW,), jnp.int32),
        pltpu.VMEM((EPW,), jnp.float32),
        pltpu.VMEM((EPW,), jnp.float32),
        pltpu.VMEM((N,), jnp.float32),
    ],
)
def _norm_kernel(src_hbm, dst_hbm, ew_hbm, dinv_hbm, out_hbm,
                 src_v, dst_v, ew_v, nrm_v, dinv_v):
    w = _worker_id()
    base = w * EPW
    pltpu.sync_copy(src_hbm.at[pl.ds(base, EPW)], src_v)
    pltpu.sync_copy(dst_hbm.at[pl.ds(base, EPW)], dst_v)
    pltpu.sync_copy(ew_hbm.at[pl.ds(base, EPW)], ew_v)
    pltpu.sync_copy(dinv_hbm, dinv_v)

    def body(g, carry):
        o = g * 16
        s = src_v[pl.ds(o, 16)]
        d = dst_v[pl.ds(o, 16)]
        ew = ew_v[pl.ds(o, 16)]
        a = plsc.load_gather(dinv_v, [s])
        b = plsc.load_gather(dinv_v, [d])
        nrm_v[pl.ds(o, 16)] = a * ew * b
        return carry

    lax.fori_loop(0, EPW // 16, body, 0)
    pltpu.sync_copy(nrm_v, out_hbm.at[pl.ds(base, EPW)])


@functools.partial(
    pl.kernel,
    out_type=jax.ShapeDtypeStruct((N, D), jnp.float32),
    mesh=_mesh,
    compiler_params=_sc_params,
    scratch_types=[
        pltpu.VMEM((CR * 128,), jnp.int32),
        pltpu.VMEM((CR * 128,), jnp.int32),
        pltpu.VMEM((CR * 128,), jnp.float32),
        pltpu.VMEM((128, D), jnp.float32),
        pltpu.VMEM((128, D), jnp.float32),
        pltpu.VMEM((128, D), jnp.float32),
        pltpu.VMEM((128, D), jnp.float32),
        pltpu.VMEM((128,), jnp.int32),
        pltpu.VMEM((128,), jnp.int32),
        pltpu.VMEM((128,), jnp.int32),
        pltpu.VMEM((128,), jnp.int32),
        pltpu.VMEM((128,), jnp.int32),
        pltpu.VMEM((128,), jnp.int32),
        pltpu.VMEM((128,), jnp.int32),
        pltpu.VMEM((128,), jnp.int32),
        pltpu.VMEM_SHARED((ACCR, D), jnp.float32),
        pltpu.SemaphoreType.DMA,
        pltpu.SemaphoreType.DMA,
        pltpu.SemaphoreType.DMA,
        pltpu.SemaphoreType.DMA,
        pltpu.SemaphoreType.DMA,
        pltpu.SemaphoreType.DMA,
        pltpu.SemaphoreType.DMA,
        pltpu.SemaphoreType.DMA,
    ],
)
def _msgpass_kernel(hw_hbm, src_hbm, dst_hbm, nrm_hbm, out_hbm,
                    sbuf, dbuf, nbuf, rb0, rb1, rb2, rb3,
                    se0, se1, se2, se3, de0, de1, de2, de3,
                    acc, g0, g1, g2, g3, s0, s1, s2, s3):
    c = lax.axis_index("c")
    s = lax.axis_index("s")

    rbufs = (rb0, rb1, rb2, rb3)
    gidx = (se0, se1, se2, se3)
    didx = (de0, de1, de2, de3)
    gsems = (g0, g1, g2, g3)
    ssems = (s0, s1, s2, s3)
    RING = 4

    zeros = jnp.zeros((16,), jnp.float32)

    def zbody(i, carry):
        for k8 in range(8):
            rb0[i, pl.ds(k8 * 16, 16)] = zeros
        return carry

    lax.fori_loop(0, 128, zbody, 0)
    for j in range(3):
        pltpu.sync_copy(rb0, acc.at[pl.ds(s * 312 + j * 128, 128)])
    plsc.subcore_barrier()

    def gstart(r, k):
        for j8 in range(8):
            gidx[k][pl.ds(j8 * 16, 16)] = (
                sbuf[pl.ds(r * 128 + j8 * 16, 16)])
        pltpu.async_copy(hw_hbm.at[gidx[k]], rbufs[k], gsems[k])

    def gwait(k):
        pltpu.make_async_copy(hw_hbm.at[gidx[k]], rbufs[k], gsems[k]).wait()

    def sstart(r, k):
        for j8 in range(8):
            d = dbuf[pl.ds(r * 128 + j8 * 16, 16)]
            dloc = d - c * NH
            ok = (dloc >= 0) & (dloc < NH)
            didx[k][pl.ds(j8 * 16, 16)] = jnp.where(ok, dloc, NH + (d & 63))
        pltpu.async_copy(rbufs[k], acc.at[didx[k]], ssems[k], add=True)

    def swait(k):
        pltpu.make_async_copy(rbufs[k], acc.at[didx[k]], ssems[k]).wait()

    def scale(k, r):
        rb = rbufs[k]

        def bbody(b, carry):
            nv = nbuf[pl.ds(r * 128 + b * 16, 16)]
            e0 = b * 16
            for lane in range(16):
                sp = _splat(nv, lane)
                e = e0 + lane
                for k8 in range(8):
                    rb[e, pl.ds(k8 * 16, 16)] = rb[e, pl.ds(k8 * 16, 16)] * sp
            return carry

        lax.fori_loop(0, 8, bbody, 0)

    for cc in range(RPW // CR):
        cbase = (s * RPW + cc * CR) * 128
        pltpu.sync_copy(src_hbm.at[pl.ds(cbase, CR * 128)], sbuf)
        pltpu.sync_copy(dst_hbm.at[pl.ds(cbase, CR * 128)], dbuf)
        pltpu.sync_copy(nrm_hbm.at[pl.ds(cbase, CR * 128)], nbuf)

        for k in range(RING):
            gstart(k, k)
        for k in range(RING):
            gwait(k)
            scale(k, k)
            sstart(k, k)

        def mbody(rr, carry):
            b2 = rr * RING
            for k in range(RING):
                swait(k)
                gstart(b2 + k, k)
            for k in range(RING):
                gwait(k)
                scale(k, b2 + k)
                sstart(b2 + k, k)
            return carry

        lax.fori_loop(1, CR // RING, mbody, 0)
        for k in range(RING):
            swait(k)

    plsc.subcore_barrier()

    out_base = c * NH + s * 312
    pltpu.sync_copy(
        acc.at[pl.ds(s * 312, ZR)],
        out_hbm.at[pl.ds(out_base, ZR)])

    @pl.when(s < NS - 1)
    def _wb_small():
        pltpu.sync_copy(
            acc.at[pl.ds(s * 312 + ZR, 152)],
            out_hbm.at[pl.ds(out_base + ZR, 152)])

    @pl.when(s == NS - 1)
    def _wb_last():
        pltpu.sync_copy(
            acc.at[pl.ds(s * 312 + ZR, ZR)],
            out_hbm.at[pl.ds(out_base + ZR, ZR)])


def _prep_body(p_ref, dinv_ref, d2_ref):
    deg = jnp.sum(p_ref[...], axis=0, keepdims=True) + 1.0
    dinv = lax.rsqrt(deg)
    dinv_ref[...] = dinv
    d2_ref[...] = dinv * dinv


_prep = pl.pallas_call(
    _prep_body,
    out_shape=[
        jax.ShapeDtypeStruct((1, N), jnp.float32),
        jax.ShapeDtypeStruct((1, N), jnp.float32),
    ],
)


def _mm0_body(x_ref, w_ref, hw_ref):
    hw_ref[...] = jnp.dot(
        x_ref[...], w_ref[...], preferred_element_type=jnp.float32)


_mm0 = pl.pallas_call(
    _mm0_body,
    out_shape=jax.ShapeDtypeStruct((N, D), jnp.float32),
)


def _mmn_body(p_ref, hwp_ref, d2_ref, b_ref, w_ref, hw_ref):
    h = jnp.maximum(
        p_ref[...] + d2_ref[...] * hwp_ref[...] + b_ref[...], 0.0)
    hw_ref[...] = jnp.dot(h, w_ref[...], preferred_element_type=jnp.float32)


_mmn = pl.pallas_call(
    _mmn_body,
    out_shape=jax.ShapeDtypeStruct((N, D), jnp.float32),
)


def _final_body(p_ref, hwp_ref, d2_ref, b_ref, out_ref):
    out_ref[...] = p_ref[...] + d2_ref[...] * hwp_ref[...] + b_ref[...]


_final = pl.pallas_call(
    _final_body,
    out_shape=jax.ShapeDtypeStruct((N, D), jnp.float32),
)


def kernel(x, edge_index, edge_attr, W0, b0, W1, b1, W2, b2):
    src = edge_index[0]
    dst = edge_index[1]

    partials = _deg_kernel(dst, edge_attr)
    dinv2d, d2 = _prep(partials)
    nrm = _norm_kernel(src, dst, edge_attr, dinv2d.reshape(N))

    pad = EP - E
    src2 = jnp.concatenate([src, jnp.zeros((pad,), src.dtype)])
    dst2 = jnp.concatenate([dst, jnp.zeros((pad,), dst.dtype)])
    nrm2 = jnp.concatenate([nrm, jnp.zeros((pad,), nrm.dtype)])
    d2c = d2.reshape(N, 1)
    b0r = b0.reshape(1, D)
    b1r = b1.reshape(1, D)
    b2r = b2.reshape(1, D)

    hw0 = _mm0(x, W0)
    p = _msgpass_kernel(hw0, src2, dst2, nrm2)
    hw1 = _mmn(p, hw0, d2c, b0r, W1)
    p = _msgpass_kernel(hw1, src2, dst2, nrm2)
    hw2 = _mmn(p, hw1, d2c, b1r, W2)
    p = _msgpass_kernel(hw2, src2, dst2, nrm2)
    out = _final(p, hw2, d2c, b2r)
    return out.reshape(1, N, D)


# speedup vs baseline: 1.2524x; 1.2397x over previous
import functools

import jax
import jax.numpy as jnp
from jax import lax
from jax.experimental import pallas as pl
from jax.experimental.pallas import tpu as pltpu
from jax.experimental.pallas import tpu_sc as plsc

N = 10000
E = 320000
D = 128

NC = 2
NS = 16
NW = NC * NS
FPW = D // NW
EPW = E // NW
CE = 4000
NG = CE // 16
NCHUNK = E // CE
UNROLL = 2

_mesh = plsc.VectorSubcoreMesh(core_axis_name="c", subcore_axis_name="s")

_sc_params = pltpu.CompilerParams(needs_layout_passes=False)


def _worker_id():
    return lax.axis_index("s") * NC + lax.axis_index("c")


@functools.partial(
    pl.kernel,
    out_type=jax.ShapeDtypeStruct((NW, N), jnp.float32),
    mesh=_mesh,
    compiler_params=_sc_params,
    scratch_types=[
        pltpu.VMEM((EPW,), jnp.int32),
        pltpu.VMEM((EPW,), jnp.float32),
        pltpu.VMEM((N,), jnp.float32),
    ],
)
def _deg_kernel(dst_hbm, ew_hbm, out_hbm, dst_v, ew_v, deg_v):
    w = _worker_id()
    base = w * EPW
    pltpu.sync_copy(dst_hbm.at[pl.ds(base, EPW)], dst_v)
    pltpu.sync_copy(ew_hbm.at[pl.ds(base, EPW)], ew_v)

    zeros = jnp.zeros((16,), jnp.float32)

    def zbody(i, carry):
        deg_v[pl.ds(i * 16, 16)] = zeros
        return carry

    lax.fori_loop(0, N // 16, zbody, 0)

    def ebody(g, carry):
        o = g * 16
        d = dst_v[pl.ds(o, 16)]
        ew = ew_v[pl.ds(o, 16)]
        plsc.addupdate_scatter(deg_v, [d], ew)
        return carry

    lax.fori_loop(0, EPW // 16, ebody, 0)
    pltpu.sync_copy(deg_v, out_hbm.at[w])


@functools.partial(
    pl.kernel,
    out_type=jax.ShapeDtypeStruct((E,), jnp.float32),
    mesh=_mesh,
    compiler_params=_sc_params,
    scratch_types=[
        pltpu.VMEM((EPW,), jnp.int32),
        pltpu.VMEM((EPW,), jnp.int32),
        pltpu.VMEM((EPW,), jnp.float32),
        pltpu.VMEM((EPW,), jnp.float32),
        pltpu.VMEM((N,), jnp.float32),
    ],
)
def _norm_kernel(src_hbm, dst_hbm, ew_hbm, dinv_hbm, out_hbm,
                 src_v, dst_v, ew_v, nrm_v, dinv_v):
    w = _worker_id()
    base = w * EPW
    pltpu.sync_copy(src_hbm.at[pl.ds(base, EPW)], src_v)
    pltpu.sync_copy(dst_hbm.at[pl.ds(base, EPW)], dst_v)
    pltpu.sync_copy(ew_hbm.at[pl.ds(base, EPW)], ew_v)
    pltpu.sync_copy(dinv_hbm, dinv_v)

    def body(g, carry):
        o = g * 16
        s = src_v[pl.ds(o, 16)]
        d = dst_v[pl.ds(o, 16)]
        ew = ew_v[pl.ds(o, 16)]
        a = plsc.load_gather(dinv_v, [s])
        b = plsc.load_gather(dinv_v, [d])
        nrm_v[pl.ds(o, 16)] = a * ew * b
        return carry

    lax.fori_loop(0, EPW // 16, body, 0)
    pltpu.sync_copy(nrm_v, out_hbm.at[pl.ds(base, EPW)])


@functools.partial(
    pl.kernel,
    out_type=jax.ShapeDtypeStruct((D, N), jnp.float32),
    mesh=_mesh,
    compiler_params=_sc_params,
    scratch_types=[
        pltpu.VMEM((FPW, N), jnp.float32),
        pltpu.VMEM((FPW, N), jnp.float32),
        pltpu.VMEM((CE,), jnp.int32),
        pltpu.VMEM((CE,), jnp.int32),
        pltpu.VMEM((CE,), jnp.float32),
        pltpu.VMEM((CE,), jnp.int32),
        pltpu.VMEM((CE,), jnp.int32),
        pltpu.VMEM((CE,), jnp.float32),
        pltpu.SemaphoreType.DMA,
        pltpu.SemaphoreType.DMA,
    ],
)
def _msgpass_kernel(hwT_hbm, initT_hbm, src_hbm, dst_hbm, nrm_hbm, out_hbm,
                    ht, acc, src_v0, dst_v0, nrm_v0, src_v1, dst_v1, nrm_v1,
                    sem0, sem1):
    w = _worker_id()
    fb = w * FPW
    pltpu.sync_copy(hwT_hbm.at[pl.ds(fb, FPW)], ht)
    pltpu.sync_copy(initT_hbm.at[pl.ds(fb, FPW)], acc)

    sems = (sem0, sem1)
    bufs = ((src_v0, dst_v0, nrm_v0), (src_v1, dst_v1, nrm_v1))
    fvecs = [jnp.full((16,), f, jnp.int32) for f in range(FPW)]

    def start(c, b):
        off = pl.multiple_of(jnp.minimum(c, NCHUNK - 1) * CE, 8)
        sv, dv, nv = bufs[b]
        pltpu.async_copy(src_hbm.at[pl.ds(off, CE)], sv, sems[b])
        pltpu.async_copy(dst_hbm.at[pl.ds(off, CE)], dv, sems[b])
        pltpu.async_copy(nrm_hbm.at[pl.ds(off, CE)], nv, sems[b])

    def wait(b):
        sv, dv, nv = bufs[b]
        pltpu.make_async_copy(src_hbm.at[pl.ds(0, CE)], sv, sems[b]).wait()
        pltpu.make_async_copy(dst_hbm.at[pl.ds(0, CE)], dv, sems[b]).wait()
        pltpu.make_async_copy(nrm_hbm.at[pl.ds(0, CE)], nv, sems[b]).wait()

    def process(b):
        sv, dv, nv = bufs[b]

        def grp_body(g, c2):
            for u in range(UNROLL):
                o = (g * UNROLL + u) * 16
                s = sv[pl.ds(o, 16)]
                d = dv[pl.ds(o, 16)]
                w16 = nv[pl.ds(o, 16)]
                for f in range(FPW):
                    v = plsc.load_gather(ht, [fvecs[f], s])
                    plsc.addupdate_scatter(acc, [fvecs[f], d], v * w16)
            return c2

        lax.fori_loop(0, NG // UNROLL, grp_body, 0)

    start(0, 0)

    def chunk_body(cc, carry):
        c0 = cc * 2
        wait(0)
        start(c0 + 1, 1)
        process(0)
        wait(1)
        start(c0 + 2, 0)
        process(1)
        return carry

    lax.fori_loop(0, NCHUNK // 2, chunk_body, 0)
    wait(0)
    pltpu.sync_copy(acc, out_hbm.at[pl.ds(fb, FPW)])


def _prep_body(p_ref, dinv_ref, d2_ref):
    deg = jnp.sum(p_ref[...], axis=0, keepdims=True) + 1.0
    dinv = lax.rsqrt(deg)
    dinv_ref[...] = dinv
    d2_ref[...] = dinv * dinv


_prep = pl.pallas_call(
    _prep_body,
    out_shape=[
        jax.ShapeDtypeStruct((1, N), jnp.float32),
        jax.ShapeDtypeStruct((1, N), jnp.float32),
    ],
)


def _mm0_body(x_ref, w_ref, d2_ref, hw_ref, init_ref):
    hw = lax.dot_general(
        w_ref[...], x_ref[...], (((0,), (1,)), ((), ())),
        preferred_element_type=jnp.float32)
    hw_ref[...] = hw
    init_ref[...] = hw * d2_ref[...]


_mm0 = pl.pallas_call(
    _mm0_body,
    out_shape=[
        jax.ShapeDtypeStruct((D, N), jnp.float32),
        jax.ShapeDtypeStruct((D, N), jnp.float32),
    ],
)


def _mmn_body(acc_ref, bin_ref, w_ref, d2_ref, bout_ref, hw_ref, init_ref):
    h = jnp.maximum(acc_ref[...] + bin_ref[...], 0.0)
    hw = lax.dot_general(
        w_ref[...], h, (((0,), (0,)), ((), ())),
        preferred_element_type=jnp.float32)
    hw_ref[...] = hw
    init_ref[...] = hw * d2_ref[...] + bout_ref[...]


_mmn = pl.pallas_call(
    _mmn_body,
    out_shape=[
        jax.ShapeDtypeStruct((D, N), jnp.float32),
        jax.ShapeDtypeStruct((D, N), jnp.float32),
    ],
)


def _final_body(acc_ref, b_ref, out_ref):
    out_ref[...] = acc_ref[...] + b_ref[...]


_final = pl.pallas_call(
    _final_body,
    out_shape=jax.ShapeDtypeStruct((D, N), jnp.float32),
)


def kernel(x, edge_index, edge_attr, W0, b0, W1, b1, W2, b2):
    src = edge_index[0]
    dst = edge_index[1]
    zero_col = jnp.zeros((D, 1), jnp.float32)

    partials = _deg_kernel(dst, edge_attr)
    dinv2d, d2 = _prep(partials)
    nrm = _norm_kernel(src, dst, edge_attr, dinv2d.reshape(N))

    hw0, init0 = _mm0(x, W0, d2)
    acc1 = _msgpass_kernel(hw0, init0, src, dst, nrm)

    hw1, init1 = _mmn(acc1, b0.reshape(D, 1), W1, d2, zero_col)
    acc2 = _msgpass_kernel(hw1, init1, src, dst, nrm)

    hw2, init2 = _mmn(acc2, b1.reshape(D, 1), W2, d2, zero_col)
    acc3 = _msgpass_kernel(hw2, init2, src, dst, nrm)

    out = _final(acc3, b2.reshape(D, 1))
    return out.T.reshape(1, N, D)
